# baseline (speedup 1.0000x reference)
import functools

import jax
import jax.numpy as jnp
from jax import lax
from jax.experimental import pallas as pl

_BF = jnp.float32
_BF3 = jnp.bfloat16
_F32 = jnp.float32
_IDX4 = ((0, 0), (0, 1), (1, 0), (1, 1))
_B = 128


def _dot(a, b):
    prec = None if a.dtype == _BF3 else lax.Precision.HIGHEST
    return lax.dot_general(a, b, (((1,), (0,)), ((), ())),
                           preferred_element_type=_F32, precision=prec)


def _lrelu(x):
    return jnp.where(x >= 0, x, 0.2 * x)


def _wsum(s, km):
    return lax.dot_general(s, km, (((1,), (0,)), ((), ())),
                           preferred_element_type=_F32,
                           precision=lax.Precision.HIGHEST)


def _bn_cols(x, km, n):
    s = jnp.sum(x, axis=0, keepdims=True)
    ss = jnp.sum(x * x, axis=0, keepdims=True)
    m = _wsum(s, km) / n
    v = _wsum(ss, km) / n - m * m
    return (x - m) * lax.rsqrt(v + 1e-5)


def _shift_rows(x, off):
    if off == 0:
        return x
    k = abs(off)
    zpad = jnp.zeros((k, x.shape[1]), x.dtype)
    if off > 0:
        return jnp.concatenate([x[k:], zpad], axis=0)
    return jnp.concatenate([zpad, x[:-k]], axis=0)


def _pre_body(z_ref, fcw_ref, fcb_ref, wr_ref, o_ref):
    zb = z_ref[...].astype(_BF)
    h = _lrelu(_dot(zb, fcw_ref[...]) + fcb_ref[...])
    x = _dot(h.astype(_BF), wr_ref[...])
    m = jnp.sum(x, axis=0, keepdims=True) / _B
    v = jnp.sum(x * x, axis=0, keepdims=True) / _B - m * m
    o_ref[...] = _lrelu((x - m) * lax.rsqrt(v + 1e-5))


def _tconv_body(mmdt, out_dtype, x_ref, wt_ref, o0, o1, o2, o3, st_ref):
    xb = x_ref[...].astype(mmdt)
    outs = (o0, o1, o2, o3)
    s = jnp.zeros((1, 256), _F32)
    ss = jnp.zeros((1, 256), _F32)
    for (i, j), o in zip(_IDX4, outs):
        p = _dot(xb, wt_ref[i, j])
        s = s + jnp.sum(p, axis=0, keepdims=True)
        ss = ss + jnp.sum(p * p, axis=0, keepdims=True)
        o[...] = p.astype(out_dtype)
    st_ref[...] = jnp.concatenate([s, ss], axis=0)


def _bn_from_stats(st, km, n):
    m = _wsum(st[0:1], km) / n
    v = _wsum(st[1:2], km) / n - m * m
    return m, lax.rsqrt(v + 1e-5)


def _conv_body(D, H, n, mmdt, y_ref, yst_ref, wb_ref, km_ref, o_ref):
    m, inv = _bn_from_stats(yst_ref[...], km_ref[...], n)
    y = _lrelu((y_ref[...] - m) * inv).astype(mmdt)
    R, L = y.shape
    r = lax.broadcasted_iota(jnp.int32, (R, 1), 0)
    d = r // (H * _B)
    h = (r // _B) % H
    acc = jnp.zeros((R, L), _F32)
    for dd in (-1, 0, 1):
        for dh in (-1, 0, 1):
            off = dd * H * _B + dh * _B
            ys = _shift_rows(y, off)
            if off != 0:
                ok = ((d + dd) >= 0) & ((d + dd) < D) & \
                     ((h + dh) >= 0) & ((h + dh) < H)
                ys = jnp.where(ok, ys, jnp.zeros_like(ys))
            acc = acc + _dot(ys, wb_ref[dd + 1, dh + 1])
    o_ref[...] = _lrelu(_bn_cols(acc, km_ref[...], n))


def _c3a_body(n, yp_ref, yc_ref, yn_ref, yst_ref, wb_ref, km_ref, z_ref, st_ref):
    i = pl.program_id(0)
    CH, L = yc_ref.shape
    D = H = 16
    HB = H * _B
    m, inv = _bn_from_stats(yst_ref[...], km_ref[...], n)
    prev = _lrelu((yp_ref[...].astype(_F32) - m) * inv).astype(_BF3)
    cur = _lrelu((yc_ref[...].astype(_F32) - m) * inv).astype(_BF3)
    nxt = _lrelu((yn_ref[...].astype(_F32) - m) * inv).astype(_BF3)
    rl = lax.broadcasted_iota(jnp.int32, (CH, 1), 0)
    g = i * CH + rl
    d = g // HB
    h = (g // _B) % H
    acc = jnp.zeros((CH, L), _F32)
    for dd in (-1, 0, 1):
        for dh in (-1, 0, 1):
            off = dd * HB + dh * _B
            if off == 0:
                ys = cur
            else:
                if off > 0:
                    ys = jnp.concatenate([cur[off:], nxt[:off]], axis=0)
                else:
                    k = -off
                    ys = jnp.concatenate([prev[CH - k:], cur[:CH - k]], axis=0)
                ok = ((d + dd) >= 0) & ((d + dd) < D) & \
                     ((h + dh) >= 0) & ((h + dh) < H)
                ys = jnp.where(ok, ys, jnp.zeros_like(ys))
            acc = acc + _dot(ys, wb_ref[dd + 1, dh + 1])
    s = jnp.sum(acc, axis=0, keepdims=True)
    ss = jnp.sum(acc * acc, axis=0, keepdims=True)
    st_ref[...] = jnp.concatenate([s, ss], axis=0).reshape(1, 2, 256)
    z_ref[...] = acc.astype(_BF3)


def _c3b_body(n, z_ref, st_ref, km_ref, wf_ref, b_ref, o_ref):
    st = st_ref[...]
    s = jnp.sum(st[:, 0], axis=0, keepdims=True)
    ss = jnp.sum(st[:, 1], axis=0, keepdims=True)
    km = km_ref[...]
    m = _wsum(s, km) / n
    v = _wsum(ss, km) / n - m * m
    inv = lax.rsqrt(v + 1e-5)
    zc = z_ref[...].astype(_F32)
    a = _lrelu((zc - m) * inv).astype(_BF3)
    o = _dot(a, wf_ref[...]) + b_ref[...]
    o_ref[...] = jax.nn.sigmoid(o)


def _band_mats(Wc, W):
    q = jnp.arange(W)
    sel = jnp.stack([(q[:, None] == q[None, :] + dw).astype(_F32)
                     for dw in (-1, 0, 1)])
    bm = jnp.einsum('kpq,dekio->depiqo', sel, Wc)
    C = Wc.shape[-1]
    return bm.reshape(3, 3, W * C, W * C)


def _tconv_mats(Wt, Wp):
    q = jnp.arange(Wp)
    q2 = jnp.arange(2 * Wp)
    u = jnp.stack([(q2[None, :] == 2 * q[:, None] + z).astype(_F32)
                   for z in (0, 1)])
    tm = jnp.einsum('zpq,xyzio->xypiqo', u, Wt)
    Ci, Co = Wt.shape[-2], Wt.shape[-1]
    return tm.reshape(2, 2, Wp * Ci, 2 * Wp * Co)


def _kmat(W, C):
    return jnp.einsum('pq,io->piqo', jnp.ones((W, W), _F32),
                      jnp.eye(C, dtype=_F32)).reshape(W * C, W * C)


def _ilv(parts, Dp, Hp):
    a = jnp.stack(parts, axis=0).reshape(2, 2, Dp, Hp, _B, 256)
    return a.transpose(2, 0, 3, 1, 4, 5).reshape(4 * Dp * Hp * _B, 256)


def kernel(z, fc_W, fc_b, W_init, Wt0, Wc0, Wt1, Wc1, Wt2, Wc2, Wt3, Wc3,
           W_final, b_final):
    f32 = _F32
    sds = jax.ShapeDtypeStruct

    fcw = fc_W.astype(_BF)
    fcb = fc_b.reshape(1, -1).astype(f32)
    wr = jnp.transpose(W_init[1:, 1:, 1:], (3, 0, 1, 2, 4)) \
        .reshape(2048, 256).astype(_BF)
    t0 = _tconv_mats(Wt0, 1)
    t1 = _tconv_mats(Wt1, 2)
    t2 = _tconv_mats(Wt2, 4)
    t3 = _tconv_mats(Wt3, 8)
    b0 = _band_mats(Wc0, 2)
    b1 = _band_mats(Wc1, 4)
    b2 = _band_mats(Wc2, 8)
    b3 = _band_mats(Wc3, 16).astype(_BF3)
    km0 = _kmat(2, 128)
    km1 = _kmat(4, 64)
    km2 = _kmat(8, 32)
    km3 = _kmat(16, 16)
    wf = jnp.einsum('pq,io->piqo', jnp.eye(16, dtype=f32),
                    W_final).reshape(256, 16).astype(_BF3)
    bf2 = jnp.broadcast_to(b_final.reshape(1, 1), (1, 16)).astype(f32)

    x = pl.pallas_call(
        _pre_body,
        out_shape=sds((_B, 256), f32),
    )(z, fcw, fcb, wr)

    def tconv(inp, tm, mmdt, dt):
        r = inp.shape[0]
        res = pl.pallas_call(
            functools.partial(_tconv_body, mmdt, dt),
            out_shape=[sds((r, 256), dt)] * 4 + [sds((2, 256), f32)],
        )(inp, tm.astype(mmdt))
        return res[:4], res[4]

    def conv(inp, yst, bm, km, dim, n, mmdt):
        r = inp.shape[0]
        return pl.pallas_call(
            functools.partial(_conv_body, dim, dim, n, mmdt),
            out_shape=sds((r, 256), f32),
        )(inp, yst, bm.astype(mmdt), km)

    p0, st0 = tconv(x, t0, f32, f32)
    y0 = conv(_ilv(p0, 1, 1), st0, b0, km0, 2, 1024.0, f32)
    p1, st1 = tconv(y0, t1, f32, f32)
    y1 = conv(_ilv(p1, 2, 2), st1, b1, km1, 4, 8192.0, f32)
    p2, st2 = tconv(y1, t2, f32, f32)
    y2 = conv(_ilv(p2, 4, 4), st2, b2, km2, 8, 65536.0, _BF3)
    p3, st3 = tconv(y2, t3, _BF3, _BF3)
    y3 = _ilv(p3, 8, 8)

    CH = 4096
    nchunk = 8
    z3, st = pl.pallas_call(
        functools.partial(_c3a_body, 524288.0),
        grid=(nchunk,),
        in_specs=[
            pl.BlockSpec((CH, 256), lambda i: (jnp.maximum(i - 1, 0), 0)),
            pl.BlockSpec((CH, 256), lambda i: (i, 0)),
            pl.BlockSpec((CH, 256),
                         lambda i: (jnp.minimum(i + 1, nchunk - 1), 0)),
            pl.BlockSpec((2, 256), lambda i: (0, 0)),
            pl.BlockSpec((3, 3, 256, 256), lambda i: (0, 0, 0, 0)),
            pl.BlockSpec((256, 256), lambda i: (0, 0)),
        ],
        out_specs=[
            pl.BlockSpec((CH, 256), lambda i: (i, 0)),
            pl.BlockSpec((1, 2, 256), lambda i: (i, 0, 0)),
        ],
        out_shape=[sds((32768, 256), _BF3), sds((nchunk, 2, 256), f32)],
    )(y3, y3, y3, st3, b3, km3)

    out = pl.pallas_call(
        functools.partial(_c3b_body, 524288.0),
        grid=(nchunk,),
        in_specs=[
            pl.BlockSpec((CH, 256), lambda i: (i, 0)),
            pl.BlockSpec((nchunk, 2, 256), lambda i: (0, 0, 0)),
            pl.BlockSpec((256, 256), lambda i: (0, 0)),
            pl.BlockSpec((256, 16), lambda i: (0, 0)),
            pl.BlockSpec((1, 16), lambda i: (0, 0)),
        ],
        out_specs=pl.BlockSpec((CH, 16), lambda i: (i, 0)),
        out_shape=sds((32768, 16), f32),
    )(z3, st, km3, wf, bf2)

    return out.reshape(16, 16, _B, 16).transpose(2, 0, 1, 3) \
        .reshape(_B, 1, 16, 16, 16)


# speedup vs baseline: 1.1047x; 1.1047x over previous
import functools

import jax
import jax.numpy as jnp
from jax import lax
from jax.experimental import pallas as pl

_BF = jnp.bfloat16
_BF3 = jnp.bfloat16
_F32 = jnp.float32
_IDX4 = ((0, 0), (0, 1), (1, 0), (1, 1))
_B = 128


def _dot(a, b):
    prec = None if a.dtype == _BF3 else lax.Precision.HIGHEST
    return lax.dot_general(a, b, (((1,), (0,)), ((), ())),
                           preferred_element_type=_F32, precision=prec)


def _lrelu(x):
    return jnp.where(x >= 0, x, 0.2 * x)


def _wsum(s, km):
    return lax.dot_general(s, km, (((1,), (0,)), ((), ())),
                           preferred_element_type=_F32,
                           precision=lax.Precision.HIGHEST)


def _bn_cols(x, km, n):
    s = jnp.sum(x, axis=0, keepdims=True)
    ss = jnp.sum(x * x, axis=0, keepdims=True)
    m = _wsum(s, km) / n
    v = _wsum(ss, km) / n - m * m
    return (x - m) * lax.rsqrt(v + 1e-5)


def _shift_rows(x, off):
    if off == 0:
        return x
    k = abs(off)
    zpad = jnp.zeros((k, x.shape[1]), x.dtype)
    if off > 0:
        return jnp.concatenate([x[k:], zpad], axis=0)
    return jnp.concatenate([zpad, x[:-k]], axis=0)


def _pre_body(z_ref, fcw_ref, fcb_ref, wr_ref, o_ref):
    zb = z_ref[...].astype(_BF)
    h = _lrelu(_dot(zb, fcw_ref[...]) + fcb_ref[...])
    x = _dot(h.astype(_BF), wr_ref[...])
    m = jnp.sum(x, axis=0, keepdims=True) / _B
    v = jnp.sum(x * x, axis=0, keepdims=True) / _B - m * m
    o_ref[...] = _lrelu((x - m) * lax.rsqrt(v + 1e-5))


def _tconv_body(mmdt, out_dtype, x_ref, wt_ref, o0, o1, o2, o3, st_ref):
    xb = x_ref[...].astype(mmdt)
    outs = (o0, o1, o2, o3)
    s = jnp.zeros((1, 256), _F32)
    ss = jnp.zeros((1, 256), _F32)
    for (i, j), o in zip(_IDX4, outs):
        p = _dot(xb, wt_ref[i, j])
        s = s + jnp.sum(p, axis=0, keepdims=True)
        ss = ss + jnp.sum(p * p, axis=0, keepdims=True)
        o[...] = p.astype(out_dtype)
    st_ref[...] = jnp.concatenate([s, ss], axis=0)


def _bn_from_stats(st, km, n):
    m = _wsum(st[0:1], km) / n
    v = _wsum(st[1:2], km) / n - m * m
    return m, lax.rsqrt(v + 1e-5)


def _conv_body(D, H, n, mmdt, y_ref, yst_ref, wb_ref, km_ref, o_ref):
    m, inv = _bn_from_stats(yst_ref[...], km_ref[...], n)
    y = _lrelu((y_ref[...] - m) * inv).astype(mmdt)
    R, L = y.shape
    r = lax.broadcasted_iota(jnp.int32, (R, 1), 0)
    d = r // (H * _B)
    h = (r // _B) % H
    acc = jnp.zeros((R, L), _F32)
    for dd in (-1, 0, 1):
        for dh in (-1, 0, 1):
            off = dd * H * _B + dh * _B
            ys = _shift_rows(y, off)
            if off != 0:
                ok = ((d + dd) >= 0) & ((d + dd) < D) & \
                     ((h + dh) >= 0) & ((h + dh) < H)
                ys = jnp.where(ok, ys, jnp.zeros_like(ys))
            acc = acc + _dot(ys, wb_ref[dd + 1, dh + 1])
    o_ref[...] = _lrelu(_bn_cols(acc, km_ref[...], n))


def _c3a_body(n, yp_ref, yc_ref, yn_ref, yst_ref, wb_ref, km_ref, z_ref, st_ref):
    i = pl.program_id(0)
    CH, L = yc_ref.shape
    D = H = 16
    HB = H * _B
    m, inv = _bn_from_stats(yst_ref[...], km_ref[...], n)
    prev = _lrelu((yp_ref[...].astype(_F32) - m) * inv).astype(_BF3)
    cur = _lrelu((yc_ref[...].astype(_F32) - m) * inv).astype(_BF3)
    nxt = _lrelu((yn_ref[...].astype(_F32) - m) * inv).astype(_BF3)
    rl = lax.broadcasted_iota(jnp.int32, (CH, 1), 0)
    g = i * CH + rl
    d = g // HB
    h = (g // _B) % H
    acc = jnp.zeros((CH, L), _F32)
    for dd in (-1, 0, 1):
        for dh in (-1, 0, 1):
            off = dd * HB + dh * _B
            if off == 0:
                ys = cur
            else:
                if off > 0:
                    ys = jnp.concatenate([cur[off:], nxt[:off]], axis=0)
                else:
                    k = -off
                    ys = jnp.concatenate([prev[CH - k:], cur[:CH - k]], axis=0)
                ok = ((d + dd) >= 0) & ((d + dd) < D) & \
                     ((h + dh) >= 0) & ((h + dh) < H)
                ys = jnp.where(ok, ys, jnp.zeros_like(ys))
            acc = acc + _dot(ys, wb_ref[dd + 1, dh + 1])
    s = jnp.sum(acc, axis=0, keepdims=True)
    ss = jnp.sum(acc * acc, axis=0, keepdims=True)
    st_ref[...] = jnp.concatenate([s, ss], axis=0).reshape(1, 2, 256)
    z_ref[...] = acc.astype(_BF3)


def _c3b_body(n, z_ref, st_ref, km_ref, wf_ref, b_ref, o_ref):
    st = st_ref[...]
    s = jnp.sum(st[:, 0], axis=0, keepdims=True)
    ss = jnp.sum(st[:, 1], axis=0, keepdims=True)
    km = km_ref[...]
    m = _wsum(s, km) / n
    v = _wsum(ss, km) / n - m * m
    inv = lax.rsqrt(v + 1e-5)
    zc = z_ref[...].astype(_F32)
    a = _lrelu((zc - m) * inv).astype(_BF3)
    o = _dot(a, wf_ref[...]) + b_ref[...]
    o_ref[...] = jax.nn.sigmoid(o)


def _band_mats(Wc, W):
    q = jnp.arange(W)
    sel = jnp.stack([(q[:, None] == q[None, :] + dw).astype(_F32)
                     for dw in (-1, 0, 1)])
    bm = jnp.einsum('kpq,dekio->depiqo', sel, Wc)
    C = Wc.shape[-1]
    return bm.reshape(3, 3, W * C, W * C)


def _tconv_mats(Wt, Wp):
    q = jnp.arange(Wp)
    q2 = jnp.arange(2 * Wp)
    u = jnp.stack([(q2[None, :] == 2 * q[:, None] + z).astype(_F32)
                   for z in (0, 1)])
    tm = jnp.einsum('zpq,xyzio->xypiqo', u, Wt)
    Ci, Co = Wt.shape[-2], Wt.shape[-1]
    return tm.reshape(2, 2, Wp * Ci, 2 * Wp * Co)


def _kmat(W, C):
    return jnp.einsum('pq,io->piqo', jnp.ones((W, W), _F32),
                      jnp.eye(C, dtype=_F32)).reshape(W * C, W * C)


def _ilv(parts, Dp, Hp):
    a = jnp.stack(parts, axis=0).reshape(2, 2, Dp, Hp, _B, 256)
    return a.transpose(2, 0, 3, 1, 4, 5).reshape(4 * Dp * Hp * _B, 256)


def kernel(z, fc_W, fc_b, W_init, Wt0, Wc0, Wt1, Wc1, Wt2, Wc2, Wt3, Wc3,
           W_final, b_final):
    f32 = _F32
    sds = jax.ShapeDtypeStruct

    fcw = fc_W.astype(_BF)
    fcb = fc_b.reshape(1, -1).astype(f32)
    wr = jnp.transpose(W_init[1:, 1:, 1:], (3, 0, 1, 2, 4)) \
        .reshape(2048, 256).astype(_BF)
    t0 = _tconv_mats(Wt0, 1)
    t1 = _tconv_mats(Wt1, 2)
    t2 = _tconv_mats(Wt2, 4)
    t3 = _tconv_mats(Wt3, 8)
    b0 = _band_mats(Wc0, 2)
    b1 = _band_mats(Wc1, 4)
    b2 = _band_mats(Wc2, 8)
    b3 = _band_mats(Wc3, 16).astype(_BF3)
    km0 = _kmat(2, 128)
    km1 = _kmat(4, 64)
    km2 = _kmat(8, 32)
    km3 = _kmat(16, 16)
    wf = jnp.einsum('pq,io->piqo', jnp.eye(16, dtype=f32),
                    W_final).reshape(256, 16).astype(_BF3)
    bf2 = jnp.broadcast_to(b_final.reshape(1, 1), (1, 16)).astype(f32)

    x = pl.pallas_call(
        _pre_body,
        out_shape=sds((_B, 256), f32),
    )(z, fcw, fcb, wr)

    def tconv(inp, tm, mmdt, dt):
        r = inp.shape[0]
        res = pl.pallas_call(
            functools.partial(_tconv_body, mmdt, dt),
            out_shape=[sds((r, 256), dt)] * 4 + [sds((2, 256), f32)],
        )(inp, tm.astype(mmdt))
        return res[:4], res[4]

    def conv(inp, yst, bm, km, dim, n, mmdt):
        r = inp.shape[0]
        return pl.pallas_call(
            functools.partial(_conv_body, dim, dim, n, mmdt),
            out_shape=sds((r, 256), f32),
        )(inp, yst, bm.astype(mmdt), km)

    p0, st0 = tconv(x, t0, _BF, f32)
    y0 = conv(_ilv(p0, 1, 1), st0, b0, km0, 2, 1024.0, _BF)
    p1, st1 = tconv(y0, t1, _BF, f32)
    y1 = conv(_ilv(p1, 2, 2), st1, b1, km1, 4, 8192.0, _BF)
    p2, st2 = tconv(y1, t2, _BF, f32)
    y2 = conv(_ilv(p2, 4, 4), st2, b2, km2, 8, 65536.0, _BF3)
    p3, st3 = tconv(y2, t3, _BF3, _BF3)
    y3 = _ilv(p3, 8, 8)

    CH = 4096
    nchunk = 8
    z3, st = pl.pallas_call(
        functools.partial(_c3a_body, 524288.0),
        grid=(nchunk,),
        in_specs=[
            pl.BlockSpec((CH, 256), lambda i: (jnp.maximum(i - 1, 0), 0)),
            pl.BlockSpec((CH, 256), lambda i: (i, 0)),
            pl.BlockSpec((CH, 256),
                         lambda i: (jnp.minimum(i + 1, nchunk - 1), 0)),
            pl.BlockSpec((2, 256), lambda i: (0, 0)),
            pl.BlockSpec((3, 3, 256, 256), lambda i: (0, 0, 0, 0)),
            pl.BlockSpec((256, 256), lambda i: (0, 0)),
        ],
        out_specs=[
            pl.BlockSpec((CH, 256), lambda i: (i, 0)),
            pl.BlockSpec((1, 2, 256), lambda i: (i, 0, 0)),
        ],
        out_shape=[sds((32768, 256), _BF3), sds((nchunk, 2, 256), f32)],
    )(y3, y3, y3, st3, b3, km3)

    out = pl.pallas_call(
        functools.partial(_c3b_body, 524288.0),
        grid=(nchunk,),
        in_specs=[
            pl.BlockSpec((CH, 256), lambda i: (i, 0)),
            pl.BlockSpec((nchunk, 2, 256), lambda i: (0, 0, 0)),
            pl.BlockSpec((256, 256), lambda i: (0, 0)),
            pl.BlockSpec((256, 16), lambda i: (0, 0)),
            pl.BlockSpec((1, 16), lambda i: (0, 0)),
        ],
        out_specs=pl.BlockSpec((CH, 16), lambda i: (i, 0)),
        out_shape=sds((32768, 16), f32),
    )(z3, st, km3, wf, bf2)

    return out.reshape(16, 16, _B, 16).transpose(2, 0, 1, 3) \
        .reshape(_B, 1, 16, 16, 16)
